# baseline (speedup 1.0000x reference)
import jax
import jax.numpy as jnp
from jax.experimental import pallas as pl
from jax.experimental.pallas import tpu as pltpu

_G = 32
_XR = 262144 // _G
_ZR = 16384 // _G


def _body(y_ref, w_ref, x_ref, z_ref, xo_ref, zo_ref):
    i = pl.program_id(0)
    xo_ref[...] = x_ref[...]
    zo_ref[...] = z_ref[...]

    @pl.when(i == 0)
    def _patch():
        r = jax.lax.broadcasted_iota(jnp.int32, (16, 256), 0)
        b = x_ref[0:16, :]
        b = jnp.where(r == 10, y_ref[0, :][None, :], b)
        b = jnp.where(r == 2, y_ref[1, :][None, :], b)
        b = jnp.where(r == 1, jnp.float32(45.0), b)
        xo_ref[0:16, :] = b
        rz = jax.lax.broadcasted_iota(jnp.int32, (8, 1024), 0)
        cz = jax.lax.broadcasted_iota(jnp.int32, (8, 1024), 1)
        add = (w_ref[0] * ((rz == 1) & (cz == 3)).astype(jnp.float32)
               + w_ref[1] * ((rz == 0) & (cz == 2)).astype(jnp.float32)
               + w_ref[2] * ((rz == 0) & (cz == 1)).astype(jnp.float32))
        zo_ref[0:8, :] = z_ref[0:8, :] + add


def kernel(x, y, z, w):
    xo, zo = pl.pallas_call(
        _body,
        grid=(_G,),
        in_specs=[
            pl.BlockSpec((2, 256), lambda i: (0, 0)),
            pl.BlockSpec(memory_space=pltpu.SMEM),
            pl.BlockSpec((_XR, 256), lambda i: (i, 0)),
            pl.BlockSpec((_ZR, 1024), lambda i: (i, 0)),
        ],
        out_specs=[
            pl.BlockSpec((_XR, 256), lambda i: (i, 0)),
            pl.BlockSpec((_ZR, 1024), lambda i: (i, 0)),
        ],
        out_shape=[
            jax.ShapeDtypeStruct(x.shape, x.dtype),
            jax.ShapeDtypeStruct(z.shape, z.dtype),
        ],
        compiler_params=pltpu.CompilerParams(
            dimension_semantics=("arbitrary",)),
    )(y, w, x, z)
    return (xo, zo)
